# Initial kernel scaffold; baseline (speedup 1.0000x reference)
import jax
import jax.numpy as jnp
from jax.experimental import pallas as pl


def kernel(x, Wk, bk, Wq, bq, Wv, bv, Wp, bp, Wg1, bg1, Wg2, bg2, g1, beta1, g2, beta2):
    raise NotImplementedError("write your pallas kernel here")


# speedup vs baseline: 17.5435x; 17.5435x over previous
import functools

import jax
import jax.numpy as jnp
from jax import lax
from jax.experimental import pallas as pl
from jax.experimental.pallas import tpu as pltpu
from jax.experimental.pallas import tpu_sc as plsc

PLANE = 96
INTER = 96
HEADS = 4
OUTP = 96
TOPK = 5
F = INTER * HEADS
IDX_ROWS = 8


def _stage_a_body(xf_ref, wk_ref, bk_ref, wq_ref, bq_ref, wv_ref, bv_ref,
                  sel_ref, table_ref, qt_ref, gidx_ref):
    n = xf_ref.shape[2]
    ib = pl.program_id(0)
    xf = xf_ref[0]

    ss = jnp.sum(xf * xf, axis=0, keepdims=True)
    xn = xf / jnp.maximum(jnp.sqrt(ss), 1e-12)
    s = lax.dot_general(xn, xn, (((0,), (0,)), ((), ())),
                        preferred_element_type=jnp.float32)
    row_iota = lax.broadcasted_iota(jnp.int32, (n, n), 0)
    neg = jnp.float32(-jnp.inf)
    gidx_ref[...] = jnp.zeros((IDX_ROWS, n), jnp.int32)
    for j in range(TOPK):
        m = jnp.max(s, axis=0, keepdims=True)
        cand = jnp.where(s == m, row_iota, n)
        am = jnp.min(cand, axis=0, keepdims=True)
        gidx_ref[j:j + 1, :] = am + ib * n
        s = jnp.where(row_iota == am, neg, s)

    sel = sel_ref[...]
    kr = lax.dot_general(xf, wk_ref[...], (((0,), (1,)), ((), ())),
                         preferred_element_type=jnp.float32) + bk_ref[...]
    hn = lax.dot_general(kr * kr, sel, (((1,), (0,)), ((), ())),
                         preferred_element_type=jnp.float32)
    r = 1.0 / jnp.maximum(jnp.sqrt(hn), 1e-12)
    kr = kr * lax.dot_general(r, sel, (((1,), (1,)), ((), ())),
                              preferred_element_type=jnp.float32)
    vr = lax.dot_general(xf, wv_ref[...], (((0,), (1,)), ((), ())),
                         preferred_element_type=jnp.float32) + bv_ref[...]
    table_ref[0, :, 0:F] = kr
    table_ref[0, :, F:2 * F] = vr

    qt = lax.dot_general(wq_ref[...], xf, (((1,), (0,)), ((), ())),
                         preferred_element_type=jnp.float32) + bq_ref[...]
    hq = lax.dot_general(sel, qt * qt, (((0,), (0,)), ((), ())),
                         preferred_element_type=jnp.float32)
    rq = 1.0 / jnp.maximum(jnp.sqrt(hq), 1e-12)
    qt_ref[0] = qt * lax.dot_general(sel, rq, (((1,), (0,)), ((), ())),
                                     preferred_element_type=jnp.float32)


def _stage_c_body(table_ref, g_ref, qt_ref, wp_ref, bp_ref, wg1_ref, bg1_ref,
                  wg2_ref, bg2_ref, g1_ref, beta1_ref, g2_ref, beta2_ref,
                  out_ref, z1s_ref, z2s_ref, s1_ref, ss1_ref):
    nb = pl.num_programs(0)
    n = qt_ref.shape[2]
    ib = pl.program_id(0)
    inv6 = jnp.float32(1.0 / 6.0)
    t = table_ref[0]
    g = g_ref[0]
    kl = (t[:, 0:F] + g[:, 0:F]) * inv6
    vl = (t[:, F:2 * F] + g[:, F:2 * F]) * inv6
    p = kl * vl
    ones_col = jnp.ones((n, 1), jnp.float32)
    kv = lax.dot_general(p, ones_col, (((0,), (0,)), ((), ())),
                         preferred_element_type=jnp.float32)
    hydra = qt_ref[0] * kv
    pt = lax.dot_general(wp_ref[...], hydra, (((1,), (0,)), ((), ())),
                         preferred_element_type=jnp.float32) + bp_ref[...]
    z1 = lax.dot_general(wg1_ref[...], pt, (((1,), (0,)), ((), ())),
                         preferred_element_type=jnp.float32) + bg1_ref[...]
    z1s_ref[ib] = z1
    s1_ref[ib] = jnp.sum(z1, axis=1, keepdims=True)
    ss1_ref[ib] = jnp.sum(z1 * z1, axis=1, keepdims=True)

    @pl.when(ib == nb - 1)
    def _finalize():
        cnt = jnp.float32(nb * n)
        m1 = jnp.sum(s1_ref[...], axis=0) / cnt
        v1 = jnp.sum(ss1_ref[...], axis=0) / cnt - m1 * m1
        a1 = g1_ref[...] / jnp.sqrt(v1 + 1e-5)
        c1 = beta1_ref[...] - m1 * a1
        s2 = jnp.zeros((OUTP, 1), jnp.float32)
        ss2 = jnp.zeros((OUTP, 1), jnp.float32)
        for i in range(z1s_ref.shape[0]):
            y1 = jnp.maximum(z1s_ref[i] * a1 + c1, 0.0)
            z2 = lax.dot_general(wg2_ref[...], y1, (((1,), (0,)), ((), ())),
                                 preferred_element_type=jnp.float32) + bg2_ref[...]
            z2s_ref[i] = z2
            s2 = s2 + jnp.sum(z2, axis=1, keepdims=True)
            ss2 = ss2 + jnp.sum(z2 * z2, axis=1, keepdims=True)
        m2 = s2 / cnt
        v2 = ss2 / cnt - m2 * m2
        a2 = g2_ref[...] / jnp.sqrt(v2 + 1e-5)
        c2 = beta2_ref[...] - m2 * a2
        for i in range(z2s_ref.shape[0]):
            out_ref[i] = jnp.maximum(z2s_ref[i] * a2 + c2, 0.0)


def _sc_gather_sum(table, gidx, bn):
    info = plsc.get_sparse_core_info()
    nl = info.num_lanes
    nw = info.num_cores * info.num_subcores
    rw = bn // nw
    ch = min(rw, 16)
    d = 2 * F
    mesh = plsc.VectorSubcoreMesh(core_axis_name="c", subcore_axis_name="s")

    @functools.partial(
        pl.kernel,
        out_type=jax.ShapeDtypeStruct((bn, d), jnp.float32),
        mesh=mesh,
        scratch_types=[
            [pltpu.VMEM((ch,), jnp.int32) for _ in range(TOPK)],
            [pltpu.VMEM((ch, d), jnp.float32) for _ in range(TOPK)],
            pltpu.SemaphoreType.DMA,
        ],
    )
    def k(table_hbm, gidx_hbm, out_hbm, idx_vs, bufs, sem):
        wid = lax.axis_index("s") * info.num_cores + lax.axis_index("c")
        base = wid * rw

        def chunk(c0, carry):
            off = base + c0 * ch
            for j in range(TOPK):
                pltpu.sync_copy(gidx_hbm.at[j, pl.ds(off, ch)], idx_vs[j])
            copies = [
                pltpu.async_copy(table_hbm.at[idx_vs[j]], bufs[j], sem)
                for j in range(TOPK)
            ]
            for cp in copies:
                cp.wait()

            def body(n, carry2):
                for s in range(d // nl):
                    sl = pl.ds(s * nl, nl)
                    acc = bufs[0][n, sl]
                    for j in range(1, TOPK):
                        acc = acc + bufs[j][n, sl]
                    bufs[0][n, sl] = acc
                return carry2

            lax.fori_loop(0, ch, body, 0)
            pltpu.sync_copy(bufs[0], out_hbm.at[pl.ds(off, ch)])
            return carry

        lax.fori_loop(0, rw // ch, chunk, 0)

    return k(table, gidx)


def kernel(x, Wk, bk, Wq, bq, Wv, bv, Wp, bp, Wg1, bg1, Wg2, bg2, g1, beta1,
           g2, beta2):
    b, c, h, w = x.shape
    n = h * w
    bn = b * n
    xf = x.reshape(b, c, n)
    sel = (jnp.arange(F, dtype=jnp.int32)[:, None] // INTER
           == jnp.arange(HEADS, dtype=jnp.int32)[None, :]).astype(jnp.float32)

    full = lambda s: pl.BlockSpec(s, lambda i: (0,) * len(s))
    table, qt, gidx = pl.pallas_call(
        _stage_a_body,
        grid=(b,),
        in_specs=[
            pl.BlockSpec((1, c, n), lambda i: (i, 0, 0)),
            full((F, PLANE)), full((1, F)),
            full((F, PLANE)), full((F, 1)),
            full((F, PLANE)), full((1, F)),
            full((F, HEADS)),
        ],
        out_specs=[
            pl.BlockSpec((1, n, 2 * F), lambda i: (i, 0, 0)),
            pl.BlockSpec((1, F, n), lambda i: (i, 0, 0)),
            pl.BlockSpec((IDX_ROWS, n), lambda i: (0, i)),
        ],
        out_shape=[
            jax.ShapeDtypeStruct((b, n, 2 * F), jnp.float32),
            jax.ShapeDtypeStruct((b, F, n), jnp.float32),
            jax.ShapeDtypeStruct((IDX_ROWS, bn), jnp.int32),
        ],
    )(xf, Wk, bk.reshape(1, F), Wq, bq.reshape(F, 1), Wv, bv.reshape(1, F),
      sel)

    gsum = _sc_gather_sum(table.reshape(bn, 2 * F), gidx, bn)

    out = pl.pallas_call(
        _stage_c_body,
        grid=(b,),
        in_specs=[
            pl.BlockSpec((1, n, 2 * F), lambda i: (i, 0, 0)),
            pl.BlockSpec((1, n, 2 * F), lambda i: (i, 0, 0)),
            pl.BlockSpec((1, F, n), lambda i: (i, 0, 0)),
            full((OUTP, F)), full((OUTP, 1)),
            full((OUTP, OUTP)), full((OUTP, 1)),
            full((OUTP, OUTP)), full((OUTP, 1)),
            full((OUTP, 1)), full((OUTP, 1)), full((OUTP, 1)), full((OUTP, 1)),
        ],
        out_specs=pl.BlockSpec((b, OUTP, n), lambda i: (0, 0, 0)),
        out_shape=jax.ShapeDtypeStruct((b, OUTP, n), jnp.float32),
        scratch_shapes=[
            pltpu.VMEM((b, OUTP, n), jnp.float32),
            pltpu.VMEM((b, OUTP, n), jnp.float32),
            pltpu.VMEM((b, OUTP, 1), jnp.float32),
            pltpu.VMEM((b, OUTP, 1), jnp.float32),
        ],
    )(table, gsum.reshape(b, n, 2 * F), qt, Wp, bp.reshape(OUTP, 1),
      Wg1, bg1.reshape(OUTP, 1), Wg2, bg2.reshape(OUTP, 1),
      g1.reshape(OUTP, 1), beta1.reshape(OUTP, 1), g2.reshape(OUTP, 1),
      beta2.reshape(OUTP, 1))
    return out.reshape(b, OUTP, h, w)


# speedup vs baseline: 21.7591x; 1.2403x over previous
import functools

import jax
import jax.numpy as jnp
from jax import lax
from jax.experimental import pallas as pl
from jax.experimental.pallas import tpu as pltpu
from jax.experimental.pallas import tpu_sc as plsc

PLANE = 96
INTER = 96
HEADS = 4
OUTP = 96
TOPK = 5
F = INTER * HEADS
IDX_ROWS = 8


def _stage_a_body(xf_ref, wk_ref, bk_ref, wq_ref, bq_ref, wv_ref, bv_ref,
                  sel_ref, table_ref, qt_ref, gidx_ref):
    n = xf_ref.shape[2]
    ib = pl.program_id(0)
    xf = xf_ref[0]

    ss = jnp.sum(xf * xf, axis=0, keepdims=True)
    xn = xf / jnp.maximum(jnp.sqrt(ss), 1e-12)
    s = lax.dot_general(xn, xn, (((0,), (0,)), ((), ())),
                        preferred_element_type=jnp.float32)
    row_iota = lax.broadcasted_iota(jnp.int32, (n, n), 0)
    neg = jnp.float32(-jnp.inf)
    gidx_ref[...] = jnp.zeros((IDX_ROWS, n), jnp.int32)
    for j in range(TOPK):
        m = jnp.max(s, axis=0, keepdims=True)
        cand = jnp.where(s == m, row_iota, n)
        am = jnp.min(cand, axis=0, keepdims=True)
        gidx_ref[j:j + 1, :] = am + ib * n
        s = jnp.where(row_iota == am, neg, s)

    sel = sel_ref[...]
    kr = lax.dot_general(xf, wk_ref[...], (((0,), (1,)), ((), ())),
                         preferred_element_type=jnp.float32) + bk_ref[...]
    hn = lax.dot_general(kr * kr, sel, (((1,), (0,)), ((), ())),
                         preferred_element_type=jnp.float32)
    r = 1.0 / jnp.maximum(jnp.sqrt(hn), 1e-12)
    kr = kr * lax.dot_general(r, sel, (((1,), (1,)), ((), ())),
                              preferred_element_type=jnp.float32)
    vr = lax.dot_general(xf, wv_ref[...], (((0,), (1,)), ((), ())),
                         preferred_element_type=jnp.float32) + bv_ref[...]
    table_ref[0, :, 0:F] = kr
    table_ref[0, :, F:2 * F] = vr

    qt = lax.dot_general(wq_ref[...], xf, (((1,), (0,)), ((), ())),
                         preferred_element_type=jnp.float32) + bq_ref[...]
    hq = lax.dot_general(sel, qt * qt, (((0,), (0,)), ((), ())),
                         preferred_element_type=jnp.float32)
    rq = 1.0 / jnp.maximum(jnp.sqrt(hq), 1e-12)
    qt_ref[0] = qt * lax.dot_general(sel, rq, (((1,), (0,)), ((), ())),
                                     preferred_element_type=jnp.float32)


def _stage_c_body(table_ref, g_ref, qt_ref, wp_ref, bp_ref, wg1_ref, bg1_ref,
                  wg2_ref, bg2_ref, g1_ref, beta1_ref, g2_ref, beta2_ref,
                  out_ref, z1s_ref, z2s_ref, s1_ref, ss1_ref):
    nb = pl.num_programs(0)
    n = qt_ref.shape[2]
    ib = pl.program_id(0)
    inv6 = jnp.float32(1.0 / 6.0)
    t = table_ref[0]
    g = g_ref[0]
    kl = (t[:, 0:F] + g[:, 0:F]) * inv6
    vl = (t[:, F:2 * F] + g[:, F:2 * F]) * inv6
    p = kl * vl
    ones_col = jnp.ones((n, 1), jnp.float32)
    kv = lax.dot_general(p, ones_col, (((0,), (0,)), ((), ())),
                         preferred_element_type=jnp.float32)
    hydra = qt_ref[0] * kv
    pt = lax.dot_general(wp_ref[...], hydra, (((1,), (0,)), ((), ())),
                         preferred_element_type=jnp.float32) + bp_ref[...]
    z1 = lax.dot_general(wg1_ref[...], pt, (((1,), (0,)), ((), ())),
                         preferred_element_type=jnp.float32) + bg1_ref[...]
    z1s_ref[ib] = z1
    s1_ref[ib] = jnp.sum(z1, axis=1, keepdims=True)
    ss1_ref[ib] = jnp.sum(z1 * z1, axis=1, keepdims=True)

    @pl.when(ib == nb - 1)
    def _finalize():
        cnt = jnp.float32(nb * n)
        m1 = jnp.sum(s1_ref[...], axis=0) / cnt
        v1 = jnp.sum(ss1_ref[...], axis=0) / cnt - m1 * m1
        a1 = g1_ref[...] / jnp.sqrt(v1 + 1e-5)
        c1 = beta1_ref[...] - m1 * a1
        s2 = jnp.zeros((OUTP, 1), jnp.float32)
        ss2 = jnp.zeros((OUTP, 1), jnp.float32)
        for i in range(z1s_ref.shape[0]):
            y1 = jnp.maximum(z1s_ref[i] * a1 + c1, 0.0)
            z2 = lax.dot_general(wg2_ref[...], y1, (((1,), (0,)), ((), ())),
                                 preferred_element_type=jnp.float32) + bg2_ref[...]
            z2s_ref[i] = z2
            s2 = s2 + jnp.sum(z2, axis=1, keepdims=True)
            ss2 = ss2 + jnp.sum(z2 * z2, axis=1, keepdims=True)
        m2 = s2 / cnt
        v2 = ss2 / cnt - m2 * m2
        a2 = g2_ref[...] / jnp.sqrt(v2 + 1e-5)
        c2 = beta2_ref[...] - m2 * a2
        for i in range(z2s_ref.shape[0]):
            out_ref[i] = jnp.maximum(z2s_ref[i] * a2 + c2, 0.0)


def _sc_gather_sum(table, gidx, bn):
    info = plsc.get_sparse_core_info()
    nl = info.num_lanes
    nw = info.num_cores * info.num_subcores
    rw = bn // nw
    ch = 8
    nch = rw // ch
    d = 2 * F
    mesh = plsc.VectorSubcoreMesh(core_axis_name="c", subcore_axis_name="s")

    @functools.partial(
        pl.kernel,
        out_type=jax.ShapeDtypeStruct((bn, d), jnp.float32),
        mesh=mesh,
        scratch_types=[
            pltpu.VMEM((TOPK, rw), jnp.int32),
            [[pltpu.VMEM((ch, d), jnp.float32) for _ in range(TOPK)]
             for _ in range(2)],
            [pltpu.SemaphoreType.DMA for _ in range(2)],
            [pltpu.SemaphoreType.DMA for _ in range(2)],
        ],
    )
    def k(table_hbm, gidx_hbm, out_hbm, idx_v, bufs, gsems, osems):
        wid = lax.axis_index("s") * info.num_cores + lax.axis_index("c")
        base = wid * rw
        pltpu.sync_copy(gidx_hbm.at[pl.ds(0, TOPK), pl.ds(base, rw)], idx_v)

        def issue(c):
            s = c & 1
            return [
                pltpu.async_copy(
                    table_hbm.at[idx_v.at[j, pl.ds(c * ch, ch)]],
                    bufs[s][j], gsems[s])
                for j in range(TOPK)
            ]

        def add_pass(s):
            def node(ni, carry):
                def lane(si, carry2):
                    sl = pl.ds(si * nl, nl)
                    acc = bufs[s][0][ni, sl]
                    for j in range(1, TOPK):
                        acc = acc + bufs[s][j][ni, sl]
                    bufs[s][0][ni, sl] = acc
                    return carry2
                return lax.fori_loop(0, d // nl, lane, carry)
            lax.fori_loop(0, ch, node, 0)

        store_pending = [None, None]
        gath_pending = {0: issue(0)}
        for c in range(nch):
            s = c & 1
            if c + 1 < nch:
                if store_pending[1 - s] is not None:
                    store_pending[1 - s].wait()
                    store_pending[1 - s] = None
                gath_pending[c + 1] = issue(c + 1)
            for cp in gath_pending.pop(c):
                cp.wait()
            add_pass(s)
            store_pending[s] = pltpu.async_copy(
                bufs[s][0], out_hbm.at[pl.ds(base + c * ch, ch)], osems[s])
        for sp in store_pending:
            if sp is not None:
                sp.wait()

    return k(table, gidx)


def kernel(x, Wk, bk, Wq, bq, Wv, bv, Wp, bp, Wg1, bg1, Wg2, bg2, g1, beta1,
           g2, beta2):
    b, c, h, w = x.shape
    n = h * w
    bn = b * n
    xf = x.reshape(b, c, n)
    sel = (jnp.arange(F, dtype=jnp.int32)[:, None] // INTER
           == jnp.arange(HEADS, dtype=jnp.int32)[None, :]).astype(jnp.float32)

    full = lambda s: pl.BlockSpec(s, lambda i: (0,) * len(s))
    table, qt, gidx = pl.pallas_call(
        _stage_a_body,
        grid=(b,),
        in_specs=[
            pl.BlockSpec((1, c, n), lambda i: (i, 0, 0)),
            full((F, PLANE)), full((1, F)),
            full((F, PLANE)), full((F, 1)),
            full((F, PLANE)), full((1, F)),
            full((F, HEADS)),
        ],
        out_specs=[
            pl.BlockSpec((1, n, 2 * F), lambda i: (i, 0, 0)),
            pl.BlockSpec((1, F, n), lambda i: (i, 0, 0)),
            pl.BlockSpec((IDX_ROWS, n), lambda i: (0, i)),
        ],
        out_shape=[
            jax.ShapeDtypeStruct((b, n, 2 * F), jnp.float32),
            jax.ShapeDtypeStruct((b, F, n), jnp.float32),
            jax.ShapeDtypeStruct((IDX_ROWS, bn), jnp.int32),
        ],
    )(xf, Wk, bk.reshape(1, F), Wq, bq.reshape(F, 1), Wv, bv.reshape(1, F),
      sel)

    gsum = _sc_gather_sum(table.reshape(bn, 2 * F), gidx, bn)

    out = pl.pallas_call(
        _stage_c_body,
        grid=(b,),
        in_specs=[
            pl.BlockSpec((1, n, 2 * F), lambda i: (i, 0, 0)),
            pl.BlockSpec((1, n, 2 * F), lambda i: (i, 0, 0)),
            pl.BlockSpec((1, F, n), lambda i: (i, 0, 0)),
            full((OUTP, F)), full((OUTP, 1)),
            full((OUTP, OUTP)), full((OUTP, 1)),
            full((OUTP, OUTP)), full((OUTP, 1)),
            full((OUTP, 1)), full((OUTP, 1)), full((OUTP, 1)), full((OUTP, 1)),
        ],
        out_specs=pl.BlockSpec((b, OUTP, n), lambda i: (0, 0, 0)),
        out_shape=jax.ShapeDtypeStruct((b, OUTP, n), jnp.float32),
        scratch_shapes=[
            pltpu.VMEM((b, OUTP, n), jnp.float32),
            pltpu.VMEM((b, OUTP, n), jnp.float32),
            pltpu.VMEM((b, OUTP, 1), jnp.float32),
            pltpu.VMEM((b, OUTP, 1), jnp.float32),
        ],
    )(table, gsum.reshape(b, n, 2 * F), qt, Wp, bp.reshape(OUTP, 1),
      Wg1, bg1.reshape(OUTP, 1), Wg2, bg2.reshape(OUTP, 1),
      g1.reshape(OUTP, 1), beta1.reshape(OUTP, 1), g2.reshape(OUTP, 1),
      beta2.reshape(OUTP, 1))
    return out.reshape(b, OUTP, h, w)


# speedup vs baseline: 23.1834x; 1.0655x over previous
import functools

import jax
import jax.numpy as jnp
from jax import lax
from jax.experimental import pallas as pl
from jax.experimental.pallas import tpu as pltpu
from jax.experimental.pallas import tpu_sc as plsc

PLANE = 96
INTER = 96
HEADS = 4
OUTP = 96
TOPK = 5
F = INTER * HEADS
IDX_ROWS = 8


def _stage_a_body(xf_ref, wk_ref, bk_ref, wq_ref, bq_ref, wv_ref, bv_ref,
                  sel_ref, table_ref, qt_ref, gidx_ref):
    n = xf_ref.shape[2]
    ib = pl.program_id(0)
    xf = xf_ref[0]

    ss = jnp.sum(xf * xf, axis=0, keepdims=True)
    xn = xf / jnp.maximum(jnp.sqrt(ss), 1e-12)
    s = lax.dot_general(xn, xn, (((0,), (0,)), ((), ())),
                        preferred_element_type=jnp.float32)
    row_iota = lax.broadcasted_iota(jnp.int32, (n, n), 0)
    vbits = lax.bitcast_convert_type(s, jnp.int32)
    imin = jnp.int32(-2**31)
    key = jnp.where(vbits < 0, imin - vbits, vbits)
    key = (key & jnp.int32(~(n - 1))) | (jnp.int32(n - 1) - row_iota)
    gidx_ref[...] = jnp.zeros((IDX_ROWS, n), jnp.int32)
    for j in range(TOPK):
        m = jnp.max(key, axis=0, keepdims=True)
        gidx_ref[j:j + 1, :] = (jnp.int32(n - 1) - (m & jnp.int32(n - 1))
                                + ib * n)
        key = jnp.where(key == m, imin, key)

    sel = sel_ref[...]
    kr = lax.dot_general(xf, wk_ref[...], (((0,), (1,)), ((), ())),
                         preferred_element_type=jnp.float32) + bk_ref[...]
    hn = lax.dot_general(kr * kr, sel, (((1,), (0,)), ((), ())),
                         preferred_element_type=jnp.float32)
    r = 1.0 / jnp.maximum(jnp.sqrt(hn), 1e-12)
    kr = kr * lax.dot_general(r, sel, (((1,), (1,)), ((), ())),
                              preferred_element_type=jnp.float32)
    vr = lax.dot_general(xf, wv_ref[...], (((0,), (1,)), ((), ())),
                         preferred_element_type=jnp.float32) + bv_ref[...]
    table_ref[0, :, 0:F] = kr
    table_ref[0, :, F:2 * F] = vr

    qt = lax.dot_general(wq_ref[...], xf, (((1,), (0,)), ((), ())),
                         preferred_element_type=jnp.float32) + bq_ref[...]
    hq = lax.dot_general(sel, qt * qt, (((0,), (0,)), ((), ())),
                         preferred_element_type=jnp.float32)
    rq = 1.0 / jnp.maximum(jnp.sqrt(hq), 1e-12)
    qt_ref[0] = qt * lax.dot_general(sel, rq, (((1,), (0,)), ((), ())),
                                     preferred_element_type=jnp.float32)


def _stage_c_body(table_ref, g_ref, qt_ref, wp_ref, bp_ref, wg1_ref, bg1_ref,
                  wg2_ref, bg2_ref, g1_ref, beta1_ref, g2_ref, beta2_ref,
                  out_ref, z1s_ref, z2s_ref, s1_ref, ss1_ref):
    nb = pl.num_programs(0)
    n = qt_ref.shape[2]
    ib = pl.program_id(0)
    inv6 = jnp.float32(1.0 / 6.0)
    t = table_ref[0]
    g = g_ref[0]
    kl = (t[:, 0:F] + g[:, 0:F]) * inv6
    vl = (t[:, F:2 * F] + g[:, F:2 * F]) * inv6
    p = kl * vl
    ones_col = jnp.ones((n, 1), jnp.float32)
    kv = lax.dot_general(p, ones_col, (((0,), (0,)), ((), ())),
                         preferred_element_type=jnp.float32)
    hydra = qt_ref[0] * kv
    pt = lax.dot_general(wp_ref[...], hydra, (((1,), (0,)), ((), ())),
                         preferred_element_type=jnp.float32) + bp_ref[...]
    z1 = lax.dot_general(wg1_ref[...], pt, (((1,), (0,)), ((), ())),
                         preferred_element_type=jnp.float32) + bg1_ref[...]
    z1s_ref[ib] = z1
    s1_ref[ib] = jnp.sum(z1, axis=1, keepdims=True)
    ss1_ref[ib] = jnp.sum(z1 * z1, axis=1, keepdims=True)

    @pl.when(ib == nb - 1)
    def _finalize():
        cnt = jnp.float32(nb * n)
        m1 = jnp.sum(s1_ref[...], axis=0) / cnt
        v1 = jnp.sum(ss1_ref[...], axis=0) / cnt - m1 * m1
        a1 = g1_ref[...] / jnp.sqrt(v1 + 1e-5)
        c1 = beta1_ref[...] - m1 * a1
        s2 = jnp.zeros((OUTP, 1), jnp.float32)
        ss2 = jnp.zeros((OUTP, 1), jnp.float32)
        for i in range(z1s_ref.shape[0]):
            y1 = jnp.maximum(z1s_ref[i] * a1 + c1, 0.0)
            z2 = lax.dot_general(wg2_ref[...], y1, (((1,), (0,)), ((), ())),
                                 preferred_element_type=jnp.float32) + bg2_ref[...]
            z2s_ref[i] = z2
            s2 = s2 + jnp.sum(z2, axis=1, keepdims=True)
            ss2 = ss2 + jnp.sum(z2 * z2, axis=1, keepdims=True)
        m2 = s2 / cnt
        v2 = ss2 / cnt - m2 * m2
        a2 = g2_ref[...] / jnp.sqrt(v2 + 1e-5)
        c2 = beta2_ref[...] - m2 * a2
        for i in range(z2s_ref.shape[0]):
            out_ref[i] = jnp.maximum(z2s_ref[i] * a2 + c2, 0.0)


def _sc_gather_sum(table, gidx, bn):
    info = plsc.get_sparse_core_info()
    nl = info.num_lanes
    nw = info.num_cores * info.num_subcores
    rw = bn // nw
    ch = 8
    nch = rw // ch
    d = 2 * F
    mesh = plsc.VectorSubcoreMesh(core_axis_name="c", subcore_axis_name="s")

    @functools.partial(
        pl.kernel,
        out_type=jax.ShapeDtypeStruct((bn, d), jnp.float32),
        mesh=mesh,
        scratch_types=[
            pltpu.VMEM((TOPK, rw), jnp.int32),
            [[pltpu.VMEM((ch, d), jnp.float32) for _ in range(TOPK)]
             for _ in range(2)],
            [pltpu.SemaphoreType.DMA for _ in range(2)],
            [pltpu.SemaphoreType.DMA for _ in range(2)],
        ],
    )
    def k(table_hbm, gidx_hbm, out_hbm, idx_v, bufs, gsems, osems):
        wid = lax.axis_index("s") * info.num_cores + lax.axis_index("c")
        base = wid * rw
        pltpu.sync_copy(gidx_hbm.at[pl.ds(0, TOPK), pl.ds(base, rw)], idx_v)

        def issue(c):
            s = c & 1
            return [
                pltpu.async_copy(
                    table_hbm.at[idx_v.at[j, pl.ds(c * ch, ch)]],
                    bufs[s][j], gsems[s])
                for j in range(TOPK)
            ]

        def add_pass(s):
            def node(ni, carry):
                def lane(si, carry2):
                    sl = pl.ds(si * nl, nl)
                    acc = bufs[s][0][ni, sl]
                    for j in range(1, TOPK):
                        acc = acc + bufs[s][j][ni, sl]
                    bufs[s][0][ni, sl] = acc
                    return carry2
                return lax.fori_loop(0, d // nl, lane, carry)
            lax.fori_loop(0, ch, node, 0)

        store_pending = [None, None]
        gath_pending = {0: issue(0)}
        for c in range(nch):
            s = c & 1
            if c + 1 < nch:
                if store_pending[1 - s] is not None:
                    store_pending[1 - s].wait()
                    store_pending[1 - s] = None
                gath_pending[c + 1] = issue(c + 1)
            for cp in gath_pending.pop(c):
                cp.wait()
            add_pass(s)
            store_pending[s] = pltpu.async_copy(
                bufs[s][0], out_hbm.at[pl.ds(base + c * ch, ch)], osems[s])
        for sp in store_pending:
            if sp is not None:
                sp.wait()

    return k(table, gidx)


def kernel(x, Wk, bk, Wq, bq, Wv, bv, Wp, bp, Wg1, bg1, Wg2, bg2, g1, beta1,
           g2, beta2):
    b, c, h, w = x.shape
    n = h * w
    bn = b * n
    xf = x.reshape(b, c, n)
    sel = (jnp.arange(F, dtype=jnp.int32)[:, None] // INTER
           == jnp.arange(HEADS, dtype=jnp.int32)[None, :]).astype(jnp.float32)

    full = lambda s: pl.BlockSpec(s, lambda i: (0,) * len(s))
    table, qt, gidx = pl.pallas_call(
        _stage_a_body,
        grid=(b,),
        in_specs=[
            pl.BlockSpec((1, c, n), lambda i: (i, 0, 0)),
            full((F, PLANE)), full((1, F)),
            full((F, PLANE)), full((F, 1)),
            full((F, PLANE)), full((1, F)),
            full((F, HEADS)),
        ],
        out_specs=[
            pl.BlockSpec((1, n, 2 * F), lambda i: (i, 0, 0)),
            pl.BlockSpec((1, F, n), lambda i: (i, 0, 0)),
            pl.BlockSpec((IDX_ROWS, n), lambda i: (0, i)),
        ],
        out_shape=[
            jax.ShapeDtypeStruct((b, n, 2 * F), jnp.float32),
            jax.ShapeDtypeStruct((b, F, n), jnp.float32),
            jax.ShapeDtypeStruct((IDX_ROWS, bn), jnp.int32),
        ],
    )(xf, Wk, bk.reshape(1, F), Wq, bq.reshape(F, 1), Wv, bv.reshape(1, F),
      sel)

    gsum = _sc_gather_sum(table.reshape(bn, 2 * F), gidx, bn)

    out = pl.pallas_call(
        _stage_c_body,
        grid=(b,),
        in_specs=[
            pl.BlockSpec((1, n, 2 * F), lambda i: (i, 0, 0)),
            pl.BlockSpec((1, n, 2 * F), lambda i: (i, 0, 0)),
            pl.BlockSpec((1, F, n), lambda i: (i, 0, 0)),
            full((OUTP, F)), full((OUTP, 1)),
            full((OUTP, OUTP)), full((OUTP, 1)),
            full((OUTP, OUTP)), full((OUTP, 1)),
            full((OUTP, 1)), full((OUTP, 1)), full((OUTP, 1)), full((OUTP, 1)),
        ],
        out_specs=pl.BlockSpec((b, OUTP, n), lambda i: (0, 0, 0)),
        out_shape=jax.ShapeDtypeStruct((b, OUTP, n), jnp.float32),
        scratch_shapes=[
            pltpu.VMEM((b, OUTP, n), jnp.float32),
            pltpu.VMEM((b, OUTP, n), jnp.float32),
            pltpu.VMEM((b, OUTP, 1), jnp.float32),
            pltpu.VMEM((b, OUTP, 1), jnp.float32),
        ],
    )(table, gsum.reshape(b, n, 2 * F), qt, Wp, bp.reshape(OUTP, 1),
      Wg1, bg1.reshape(OUTP, 1), Wg2, bg2.reshape(OUTP, 1),
      g1.reshape(OUTP, 1), beta1.reshape(OUTP, 1), g2.reshape(OUTP, 1),
      beta2.reshape(OUTP, 1))
    return out.reshape(b, OUTP, h, w)


# speedup vs baseline: 28.5383x; 1.2310x over previous
import functools

import jax
import jax.numpy as jnp
from jax import lax
from jax.experimental import pallas as pl
from jax.experimental.pallas import tpu as pltpu
from jax.experimental.pallas import tpu_sc as plsc

PLANE = 96
INTER = 96
HEADS = 4
OUTP = 96
TOPK = 5
F = INTER * HEADS
IDX_ROWS = 8


def _stage_a_body(xf_ref, wk_ref, bk_ref, wq_ref, bq_ref, wv_ref, bv_ref,
                  sel_ref, table_ref, qt_ref, gidx_ref):
    n = xf_ref.shape[2]
    ib = pl.program_id(0)
    xf = xf_ref[0]

    ss = jnp.sum(xf * xf, axis=0, keepdims=True)
    xn = xf / jnp.maximum(jnp.sqrt(ss), 1e-12)
    s = lax.dot_general(xn, xn, (((0,), (0,)), ((), ())),
                        preferred_element_type=jnp.float32)
    row_iota = lax.broadcasted_iota(jnp.int32, (n, n), 0)
    vbits = lax.bitcast_convert_type(s, jnp.int32)
    imin = jnp.int32(-2**31)
    key = jnp.where(vbits < 0, imin - vbits, vbits)
    key = (key & jnp.int32(~(n - 1))) | (jnp.int32(n - 1) - row_iota)
    gidx_ref[...] = jnp.zeros((IDX_ROWS, n), jnp.int32)
    for j in range(TOPK):
        m = jnp.max(key, axis=0, keepdims=True)
        gidx_ref[j:j + 1, :] = (jnp.int32(n - 1) - (m & jnp.int32(n - 1))
                                + ib * n)
        key = jnp.where(key == m, imin, key)

    sel = sel_ref[...]
    kr = lax.dot_general(xf, wk_ref[...], (((0,), (1,)), ((), ())),
                         preferred_element_type=jnp.float32) + bk_ref[...]
    hn = lax.dot_general(kr * kr, sel, (((1,), (0,)), ((), ())),
                         preferred_element_type=jnp.float32)
    r = 1.0 / jnp.maximum(jnp.sqrt(hn), 1e-12)
    kr = kr * lax.dot_general(r, sel, (((1,), (1,)), ((), ())),
                              preferred_element_type=jnp.float32)
    vr = lax.dot_general(xf, wv_ref[...], (((0,), (1,)), ((), ())),
                         preferred_element_type=jnp.float32) + bv_ref[...]
    table_ref[0, :, 0:F] = kr
    table_ref[0, :, F:2 * F] = vr

    qt = lax.dot_general(wq_ref[...], xf, (((1,), (0,)), ((), ())),
                         preferred_element_type=jnp.float32) + bq_ref[...]
    hq = lax.dot_general(sel, qt * qt, (((0,), (0,)), ((), ())),
                         preferred_element_type=jnp.float32)
    rq = 1.0 / jnp.maximum(jnp.sqrt(hq), 1e-12)
    qt_ref[0] = qt * lax.dot_general(sel, rq, (((1,), (0,)), ((), ())),
                                     preferred_element_type=jnp.float32)


def _stage_c_body(table_ref, g_ref, qt_ref, wp_ref, bp_ref, wg1_ref, bg1_ref,
                  wg2_ref, bg2_ref, g1_ref, beta1_ref, g2_ref, beta2_ref,
                  out_ref, z1s_ref, z2s_ref, s1_ref, ss1_ref):
    nb = pl.num_programs(0)
    n = qt_ref.shape[2]
    ib = pl.program_id(0)
    inv6 = jnp.float32(1.0 / 6.0)
    t = table_ref[0]
    g = g_ref[0]
    kl = (t[:, 0:F] + g[:, 0:F]) * inv6
    vl = (t[:, F:2 * F] + g[:, F:2 * F]) * inv6
    p = kl * vl
    ones_col = jnp.ones((n, 1), jnp.float32)
    kv = lax.dot_general(p, ones_col, (((0,), (0,)), ((), ())),
                         preferred_element_type=jnp.float32)
    hydra = qt_ref[0] * kv
    pt = lax.dot_general(wp_ref[...], hydra, (((1,), (0,)), ((), ())),
                         preferred_element_type=jnp.float32) + bp_ref[...]
    z1 = lax.dot_general(wg1_ref[...], pt, (((1,), (0,)), ((), ())),
                         preferred_element_type=jnp.float32) + bg1_ref[...]
    z1s_ref[ib] = z1
    s1_ref[ib] = jnp.sum(z1, axis=1, keepdims=True)
    ss1_ref[ib] = jnp.sum(z1 * z1, axis=1, keepdims=True)

    @pl.when(ib == nb - 1)
    def _finalize():
        cnt = jnp.float32(nb * n)
        m1 = jnp.sum(s1_ref[...], axis=0) / cnt
        v1 = jnp.sum(ss1_ref[...], axis=0) / cnt - m1 * m1
        a1 = g1_ref[...] / jnp.sqrt(v1 + 1e-5)
        c1 = beta1_ref[...] - m1 * a1
        s2 = jnp.zeros((OUTP, 1), jnp.float32)
        ss2 = jnp.zeros((OUTP, 1), jnp.float32)
        for i in range(z1s_ref.shape[0]):
            y1 = jnp.maximum(z1s_ref[i] * a1 + c1, 0.0)
            z2 = lax.dot_general(wg2_ref[...], y1, (((1,), (0,)), ((), ())),
                                 preferred_element_type=jnp.float32) + bg2_ref[...]
            z2s_ref[i] = z2
            s2 = s2 + jnp.sum(z2, axis=1, keepdims=True)
            ss2 = ss2 + jnp.sum(z2 * z2, axis=1, keepdims=True)
        m2 = s2 / cnt
        v2 = ss2 / cnt - m2 * m2
        a2 = g2_ref[...] / jnp.sqrt(v2 + 1e-5)
        c2 = beta2_ref[...] - m2 * a2
        for i in range(z2s_ref.shape[0]):
            out_ref[i] = jnp.maximum(z2s_ref[i] * a2 + c2, 0.0)


def _sc_gather_sum(table, gidx, bn):
    info = plsc.get_sparse_core_info()
    nl = info.num_lanes
    nw = info.num_cores * info.num_subcores
    rw = bn // nw
    ch = 8
    nch = rw // ch
    d = 2 * F
    mesh = plsc.VectorSubcoreMesh(core_axis_name="c", subcore_axis_name="s")

    @functools.partial(
        pl.kernel,
        out_type=jax.ShapeDtypeStruct((bn, d), jnp.float32),
        mesh=mesh,
        scratch_types=[
            pltpu.VMEM((TOPK, rw), jnp.int32),
            [[pltpu.VMEM((ch, d), jnp.float32) for _ in range(TOPK)]
             for _ in range(2)],
            [pltpu.VMEM((ch, d), jnp.float32) for _ in range(2)],
            [pltpu.SemaphoreType.DMA for _ in range(2)],
            [pltpu.SemaphoreType.DMA for _ in range(2)],
        ],
    )
    def k(table_hbm, gidx_hbm, out_hbm, idx_v, bufs, obufs, gsems, osems):
        wid = lax.axis_index("s") * info.num_cores + lax.axis_index("c")
        base = wid * rw
        pltpu.sync_copy(gidx_hbm.at[pl.ds(0, TOPK), pl.ds(base, rw)], idx_v)

        def issue(c, s):
            for j in range(TOPK):
                pltpu.async_copy(
                    table_hbm.at[idx_v.at[j, pl.ds(c * ch, ch)]],
                    bufs[s][j], gsems[s])

        def drain_gathers(s):
            for j in range(TOPK):
                pltpu.make_async_copy(
                    table_hbm.at[idx_v.at[j, pl.ds(0, ch)]],
                    bufs[s][j], gsems[s]).wait()

        def wait_store(s):
            pltpu.make_async_copy(
                obufs[s], out_hbm.at[pl.ds(base, ch)], osems[s]).wait()

        def add_pass(s):
            def node(ni, carry):
                for si in range(d // nl):
                    sl = pl.ds(si * nl, nl)
                    acc = bufs[s][0][ni, sl]
                    for j in range(1, TOPK):
                        acc = acc + bufs[s][j][ni, sl]
                    obufs[s][ni, sl] = acc
                return carry
            lax.fori_loop(0, ch, node, 0)

        def body(i, carry):
            c0 = 2 * i
            drain_gathers(0)

            @pl.when(i > 0)
            def _ws0():
                wait_store(0)

            add_pass(0)

            @pl.when(c0 + 2 < nch)
            def _ig0():
                issue(c0 + 2, 0)

            pltpu.async_copy(obufs[0], out_hbm.at[pl.ds(base + c0 * ch, ch)],
                             osems[0])
            drain_gathers(1)

            @pl.when(i > 0)
            def _ws1():
                wait_store(1)

            add_pass(1)

            @pl.when(c0 + 3 < nch)
            def _ig1():
                issue(c0 + 3, 1)

            pltpu.async_copy(obufs[1], out_hbm.at[pl.ds(base + (c0 + 1) * ch,
                                                        ch)], osems[1])
            return carry

        issue(0, 0)
        issue(1, 1)
        lax.fori_loop(0, nch // 2, body, 0)
        wait_store(0)
        wait_store(1)

    return k(table, gidx)


def kernel(x, Wk, bk, Wq, bq, Wv, bv, Wp, bp, Wg1, bg1, Wg2, bg2, g1, beta1,
           g2, beta2):
    b, c, h, w = x.shape
    n = h * w
    bn = b * n
    xf = x.reshape(b, c, n)
    sel = (jnp.arange(F, dtype=jnp.int32)[:, None] // INTER
           == jnp.arange(HEADS, dtype=jnp.int32)[None, :]).astype(jnp.float32)

    full = lambda s: pl.BlockSpec(s, lambda i: (0,) * len(s))
    table, qt, gidx = pl.pallas_call(
        _stage_a_body,
        grid=(b,),
        in_specs=[
            pl.BlockSpec((1, c, n), lambda i: (i, 0, 0)),
            full((F, PLANE)), full((1, F)),
            full((F, PLANE)), full((F, 1)),
            full((F, PLANE)), full((1, F)),
            full((F, HEADS)),
        ],
        out_specs=[
            pl.BlockSpec((1, n, 2 * F), lambda i: (i, 0, 0)),
            pl.BlockSpec((1, F, n), lambda i: (i, 0, 0)),
            pl.BlockSpec((IDX_ROWS, n), lambda i: (0, i)),
        ],
        out_shape=[
            jax.ShapeDtypeStruct((b, n, 2 * F), jnp.float32),
            jax.ShapeDtypeStruct((b, F, n), jnp.float32),
            jax.ShapeDtypeStruct((IDX_ROWS, bn), jnp.int32),
        ],
    )(xf, Wk, bk.reshape(1, F), Wq, bq.reshape(F, 1), Wv, bv.reshape(1, F),
      sel)

    gsum = _sc_gather_sum(table.reshape(bn, 2 * F), gidx, bn)

    out = pl.pallas_call(
        _stage_c_body,
        grid=(b,),
        in_specs=[
            pl.BlockSpec((1, n, 2 * F), lambda i: (i, 0, 0)),
            pl.BlockSpec((1, n, 2 * F), lambda i: (i, 0, 0)),
            pl.BlockSpec((1, F, n), lambda i: (i, 0, 0)),
            full((OUTP, F)), full((OUTP, 1)),
            full((OUTP, OUTP)), full((OUTP, 1)),
            full((OUTP, OUTP)), full((OUTP, 1)),
            full((OUTP, 1)), full((OUTP, 1)), full((OUTP, 1)), full((OUTP, 1)),
        ],
        out_specs=pl.BlockSpec((b, OUTP, n), lambda i: (0, 0, 0)),
        out_shape=jax.ShapeDtypeStruct((b, OUTP, n), jnp.float32),
        scratch_shapes=[
            pltpu.VMEM((b, OUTP, n), jnp.float32),
            pltpu.VMEM((b, OUTP, n), jnp.float32),
            pltpu.VMEM((b, OUTP, 1), jnp.float32),
            pltpu.VMEM((b, OUTP, 1), jnp.float32),
        ],
    )(table, gsum.reshape(b, n, 2 * F), qt, Wp, bp.reshape(OUTP, 1),
      Wg1, bg1.reshape(OUTP, 1), Wg2, bg2.reshape(OUTP, 1),
      g1.reshape(OUTP, 1), beta1.reshape(OUTP, 1), g2.reshape(OUTP, 1),
      beta2.reshape(OUTP, 1))
    return out.reshape(b, OUTP, h, w)


# speedup vs baseline: 29.5798x; 1.0365x over previous
import functools

import jax
import jax.numpy as jnp
from jax import lax
from jax.experimental import pallas as pl
from jax.experimental.pallas import tpu as pltpu
from jax.experimental.pallas import tpu_sc as plsc

PLANE = 96
INTER = 96
HEADS = 4
OUTP = 96
TOPK = 5
F = INTER * HEADS
IDX_ROWS = 8


def _stage_a_body(xf_ref, wk_ref, bk_ref, wq_ref, bq_ref, wv_ref, bv_ref,
                  sel_ref, table_ref, qt_ref, gidx_ref):
    n = xf_ref.shape[2]
    ib = pl.program_id(0)
    xf = xf_ref[0]

    ss = jnp.sum(xf * xf, axis=0, keepdims=True)
    xn = xf / jnp.maximum(jnp.sqrt(ss), 1e-12)
    s = lax.dot_general(xn, xn, (((0,), (0,)), ((), ())),
                        preferred_element_type=jnp.float32)
    row_iota = lax.broadcasted_iota(jnp.int32, (n, n), 0)
    vbits = lax.bitcast_convert_type(s, jnp.int32)
    imin = jnp.int32(-2**31)
    key = jnp.where(vbits < 0, imin - vbits, vbits)
    key = (key & jnp.int32(~(n - 1))) | (jnp.int32(n - 1) - row_iota)
    gidx_ref[...] = jnp.zeros((IDX_ROWS, n), jnp.int32)
    for j in range(TOPK):
        m = jnp.max(key, axis=0, keepdims=True)
        gidx_ref[j:j + 1, :] = (jnp.int32(n - 1) - (m & jnp.int32(n - 1))
                                + ib * n)
        key = jnp.where(key == m, imin, key)

    sel = sel_ref[...]
    kr = lax.dot_general(xf, wk_ref[...], (((0,), (1,)), ((), ())),
                         preferred_element_type=jnp.float32) + bk_ref[...]
    hn = lax.dot_general(kr * kr, sel, (((1,), (0,)), ((), ())),
                         preferred_element_type=jnp.float32)
    r = 1.0 / jnp.maximum(jnp.sqrt(hn), 1e-12)
    kr = kr * lax.dot_general(r, sel, (((1,), (1,)), ((), ())),
                              preferred_element_type=jnp.float32)
    vr = lax.dot_general(xf, wv_ref[...], (((0,), (1,)), ((), ())),
                         preferred_element_type=jnp.float32) + bv_ref[...]
    table_ref[0, :, 0:F] = kr
    table_ref[0, :, F:2 * F] = vr

    qt = lax.dot_general(wq_ref[...], xf, (((1,), (0,)), ((), ())),
                         preferred_element_type=jnp.float32) + bq_ref[...]
    hq = lax.dot_general(sel, qt * qt, (((0,), (0,)), ((), ())),
                         preferred_element_type=jnp.float32)
    rq = 1.0 / jnp.maximum(jnp.sqrt(hq), 1e-12)
    qt_ref[0] = qt * lax.dot_general(sel, rq, (((1,), (0,)), ((), ())),
                                     preferred_element_type=jnp.float32)


def _stage_c_body(t1_ref, g1_ref, q1_ref, t2_ref, g2_ref, q2_ref,
                  wp_ref, bp_ref, wg1_ref, bg1_ref,
                  wg2_ref, bg2_ref, gm1_ref, beta1_ref, gm2_ref, beta2_ref,
                  out_ref, z1s_ref, z2s_ref, s1_ref, ss1_ref):
    nb = pl.num_programs(0)
    n = q1_ref.shape[2]
    ib = pl.program_id(0)
    inv6 = jnp.float32(1.0 / 6.0)
    for half, (t_ref, g_ref, qt_ref) in enumerate(
            ((t1_ref, g1_ref, q1_ref), (t2_ref, g2_ref, q2_ref))):
        t = t_ref[0]
        g = g_ref[0]
        kl = (t[:, 0:F] + g[:, 0:F]) * inv6
        vl = (t[:, F:2 * F] + g[:, F:2 * F]) * inv6
        p = kl * vl
        ones_col = jnp.ones((n, 1), jnp.float32)
        kv = lax.dot_general(p, ones_col, (((0,), (0,)), ((), ())),
                             preferred_element_type=jnp.float32)
        hydra = qt_ref[0] * kv
        pt = lax.dot_general(wp_ref[...], hydra, (((1,), (0,)), ((), ())),
                             preferred_element_type=jnp.float32) + bp_ref[...]
        z1 = lax.dot_general(wg1_ref[...], pt, (((1,), (0,)), ((), ())),
                             preferred_element_type=jnp.float32) + bg1_ref[...]
        slot = ib + half * nb
        z1s_ref[slot] = z1
        s1_ref[slot] = jnp.sum(z1, axis=1, keepdims=True)
        ss1_ref[slot] = jnp.sum(z1 * z1, axis=1, keepdims=True)

    @pl.when(ib == nb - 1)
    def _finalize():
        cnt = jnp.float32(2 * nb * n)
        m1 = jnp.sum(s1_ref[...], axis=0) / cnt
        v1 = jnp.sum(ss1_ref[...], axis=0) / cnt - m1 * m1
        a1 = gm1_ref[...] / jnp.sqrt(v1 + 1e-5)
        c1 = beta1_ref[...] - m1 * a1
        s2 = jnp.zeros((OUTP, 1), jnp.float32)
        ss2 = jnp.zeros((OUTP, 1), jnp.float32)
        for i in range(z1s_ref.shape[0]):
            y1 = jnp.maximum(z1s_ref[i] * a1 + c1, 0.0)
            z2 = lax.dot_general(wg2_ref[...], y1, (((1,), (0,)), ((), ())),
                                 preferred_element_type=jnp.float32) + bg2_ref[...]
            z2s_ref[i] = z2
            s2 = s2 + jnp.sum(z2, axis=1, keepdims=True)
            ss2 = ss2 + jnp.sum(z2 * z2, axis=1, keepdims=True)
        m2 = s2 / cnt
        v2 = ss2 / cnt - m2 * m2
        a2 = gm2_ref[...] / jnp.sqrt(v2 + 1e-5)
        c2 = beta2_ref[...] - m2 * a2
        for i in range(z2s_ref.shape[0]):
            out_ref[i] = jnp.maximum(z2s_ref[i] * a2 + c2, 0.0)


def _sc_gather_sum(table, gidx, bn):
    info = plsc.get_sparse_core_info()
    nl = info.num_lanes
    nw = info.num_cores * info.num_subcores
    rw = bn // nw
    ch = 8
    nch = rw // ch
    d = 2 * F
    mesh = plsc.VectorSubcoreMesh(core_axis_name="c", subcore_axis_name="s")

    @functools.partial(
        pl.kernel,
        out_type=jax.ShapeDtypeStruct((bn, d), jnp.float32),
        mesh=mesh,
        scratch_types=[
            pltpu.VMEM((TOPK, rw), jnp.int32),
            [[pltpu.VMEM((ch, d), jnp.float32) for _ in range(TOPK)]
             for _ in range(2)],
            [pltpu.VMEM((ch, d), jnp.float32) for _ in range(2)],
            [pltpu.SemaphoreType.DMA for _ in range(2)],
            [pltpu.SemaphoreType.DMA for _ in range(2)],
        ],
    )
    def k(table_hbm, gidx_hbm, out_hbm, idx_v, bufs, obufs, gsems, osems):
        wid = lax.axis_index("s") * info.num_cores + lax.axis_index("c")
        base = wid * rw
        pltpu.sync_copy(gidx_hbm.at[pl.ds(0, TOPK), pl.ds(base, rw)], idx_v)

        def issue(c, s):
            for j in range(TOPK):
                pltpu.async_copy(
                    table_hbm.at[idx_v.at[j, pl.ds(c * ch, ch)]],
                    bufs[s][j], gsems[s])

        def drain_gathers(s):
            for j in range(TOPK):
                pltpu.make_async_copy(
                    table_hbm.at[idx_v.at[j, pl.ds(0, ch)]],
                    bufs[s][j], gsems[s]).wait()

        def wait_store(s):
            pltpu.make_async_copy(
                obufs[s], out_hbm.at[pl.ds(base, ch)], osems[s]).wait()

        def add_pass(s):
            def node(ni, carry):
                for si in range(d // nl):
                    sl = pl.ds(si * nl, nl)
                    acc = bufs[s][0][ni, sl]
                    for j in range(1, TOPK):
                        acc = acc + bufs[s][j][ni, sl]
                    obufs[s][ni, sl] = acc
                return carry
            lax.fori_loop(0, ch, node, 0)

        def body(i, carry):
            c0 = 2 * i
            drain_gathers(0)

            @pl.when(i > 0)
            def _ws0():
                wait_store(0)

            add_pass(0)

            @pl.when(c0 + 2 < nch)
            def _ig0():
                issue(c0 + 2, 0)

            pltpu.async_copy(obufs[0], out_hbm.at[pl.ds(base + c0 * ch, ch)],
                             osems[0])
            drain_gathers(1)

            @pl.when(i > 0)
            def _ws1():
                wait_store(1)

            add_pass(1)

            @pl.when(c0 + 3 < nch)
            def _ig1():
                issue(c0 + 3, 1)

            pltpu.async_copy(obufs[1], out_hbm.at[pl.ds(base + (c0 + 1) * ch,
                                                        ch)], osems[1])
            return carry

        issue(0, 0)
        issue(1, 1)
        lax.fori_loop(0, nch // 2, body, 0)
        wait_store(0)
        wait_store(1)

    return k(table, gidx)


def kernel(x, Wk, bk, Wq, bq, Wv, bv, Wp, bp, Wg1, bg1, Wg2, bg2, g1, beta1,
           g2, beta2):
    b, c, h, w = x.shape
    n = h * w
    bn = b * n
    xf = x.reshape(b, c, n)
    sel = (jnp.arange(F, dtype=jnp.int32)[:, None] // INTER
           == jnp.arange(HEADS, dtype=jnp.int32)[None, :]).astype(jnp.float32)

    full = lambda s: pl.BlockSpec(s, lambda i: (0,) * len(s))
    stage_a_call = pl.pallas_call(
        _stage_a_body,
        grid=(b // 2,),
        in_specs=[
            pl.BlockSpec((1, c, n), lambda i: (i, 0, 0)),
            full((F, PLANE)), full((1, F)),
            full((F, PLANE)), full((F, 1)),
            full((F, PLANE)), full((1, F)),
            full((F, HEADS)),
        ],
        out_specs=[
            pl.BlockSpec((1, n, 2 * F), lambda i: (i, 0, 0)),
            pl.BlockSpec((1, F, n), lambda i: (i, 0, 0)),
            pl.BlockSpec((IDX_ROWS, n), lambda i: (0, i)),
        ],
        out_shape=[
            jax.ShapeDtypeStruct((b // 2, n, 2 * F), jnp.float32),
            jax.ShapeDtypeStruct((b // 2, F, n), jnp.float32),
            jax.ShapeDtypeStruct((IDX_ROWS, bn // 2), jnp.int32),
        ],
    )
    stage_a = lambda xh: stage_a_call(
        xh, Wk, bk.reshape(1, F), Wq, bq.reshape(F, 1), Wv,
        bv.reshape(1, F), sel)

    table1, qt1, gidx1 = stage_a(xf[:b // 2])
    gsum1 = _sc_gather_sum(table1.reshape(bn // 2, 2 * F), gidx1, bn // 2)
    table2, qt2, gidx2 = stage_a(xf[b // 2:])
    gsum2 = _sc_gather_sum(table2.reshape(bn // 2, 2 * F), gidx2, bn // 2)

    out = pl.pallas_call(
        _stage_c_body,
        grid=(b // 2,),
        in_specs=[
            pl.BlockSpec((1, n, 2 * F), lambda i: (i, 0, 0)),
            pl.BlockSpec((1, n, 2 * F), lambda i: (i, 0, 0)),
            pl.BlockSpec((1, F, n), lambda i: (i, 0, 0)),
            pl.BlockSpec((1, n, 2 * F), lambda i: (i, 0, 0)),
            pl.BlockSpec((1, n, 2 * F), lambda i: (i, 0, 0)),
            pl.BlockSpec((1, F, n), lambda i: (i, 0, 0)),
            full((OUTP, F)), full((OUTP, 1)),
            full((OUTP, OUTP)), full((OUTP, 1)),
            full((OUTP, OUTP)), full((OUTP, 1)),
            full((OUTP, 1)), full((OUTP, 1)), full((OUTP, 1)), full((OUTP, 1)),
        ],
        out_specs=pl.BlockSpec((b, OUTP, n), lambda i: (0, 0, 0)),
        out_shape=jax.ShapeDtypeStruct((b, OUTP, n), jnp.float32),
        scratch_shapes=[
            pltpu.VMEM((b, OUTP, n), jnp.float32),
            pltpu.VMEM((b, OUTP, n), jnp.float32),
            pltpu.VMEM((b, OUTP, 1), jnp.float32),
            pltpu.VMEM((b, OUTP, 1), jnp.float32),
        ],
    )(table1, gsum1.reshape(b // 2, n, 2 * F), qt1,
      table2, gsum2.reshape(b // 2, n, 2 * F), qt2,
      Wp, bp.reshape(OUTP, 1),
      Wg1, bg1.reshape(OUTP, 1), Wg2, bg2.reshape(OUTP, 1),
      g1.reshape(OUTP, 1), beta1.reshape(OUTP, 1), g2.reshape(OUTP, 1),
      beta2.reshape(OUTP, 1))
    return out.reshape(b, OUTP, h, w)


# speedup vs baseline: 31.3691x; 1.0605x over previous
import functools

import jax
import jax.numpy as jnp
from jax import lax
from jax.experimental import pallas as pl
from jax.experimental.pallas import tpu as pltpu
from jax.experimental.pallas import tpu_sc as plsc

PLANE = 96
INTER = 96
HEADS = 4
OUTP = 96
TOPK = 5
F = INTER * HEADS
IDX_ROWS = 8


def _stage_a_body(xf_ref, wk_ref, bk_ref, wq_ref, bq_ref, wv_ref, bv_ref,
                  sel_ref, table_ref, qt_ref, gidx_ref):
    n = xf_ref.shape[2]
    ib = pl.program_id(0)
    xf = xf_ref[0]

    ss = jnp.sum(xf * xf, axis=0, keepdims=True)
    xn = xf / jnp.maximum(jnp.sqrt(ss), 1e-12)
    s = lax.dot_general(xn, xn, (((0,), (0,)), ((), ())),
                        preferred_element_type=jnp.float32)
    row_iota = lax.broadcasted_iota(jnp.int32, (n, n), 0)
    vbits = lax.bitcast_convert_type(s, jnp.int32)
    imin = jnp.int32(-2**31)
    key = jnp.where(vbits < 0, imin - vbits, vbits)
    key = (key & jnp.int32(~(n - 1))) | (jnp.int32(n - 1) - row_iota)
    gidx_ref[...] = jnp.zeros((IDX_ROWS, n), jnp.int32)
    for j in range(TOPK):
        m = jnp.max(key, axis=0, keepdims=True)
        gidx_ref[j:j + 1, :] = (jnp.int32(n - 1) - (m & jnp.int32(n - 1))
                                + ib * n)
        key = jnp.where(key == m, imin, key)

    sel = sel_ref[...]
    kr = lax.dot_general(xf, wk_ref[...], (((0,), (1,)), ((), ())),
                         preferred_element_type=jnp.float32) + bk_ref[...]
    hn = lax.dot_general(kr * kr, sel, (((1,), (0,)), ((), ())),
                         preferred_element_type=jnp.float32)
    r = 1.0 / jnp.maximum(jnp.sqrt(hn), 1e-12)
    kr = kr * lax.dot_general(r, sel, (((1,), (1,)), ((), ())),
                              preferred_element_type=jnp.float32)
    vr = lax.dot_general(xf, wv_ref[...], (((0,), (1,)), ((), ())),
                         preferred_element_type=jnp.float32) + bv_ref[...]
    table_ref[0, :, 0:F] = kr
    table_ref[0, :, F:2 * F] = vr

    qt = lax.dot_general(wq_ref[...], xf, (((1,), (0,)), ((), ())),
                         preferred_element_type=jnp.float32) + bq_ref[...]
    hq = lax.dot_general(sel, qt * qt, (((0,), (0,)), ((), ())),
                         preferred_element_type=jnp.float32)
    rq = 1.0 / jnp.maximum(jnp.sqrt(hq), 1e-12)
    qt_ref[0] = qt * lax.dot_general(sel, rq, (((1,), (0,)), ((), ())),
                                     preferred_element_type=jnp.float32)


def _z1_of(t_ref, g_ref, qt_ref, wp_ref, bp_ref, wg1_ref, bg1_ref, n):
    inv6 = jnp.float32(1.0 / 6.0)
    t = t_ref[0]
    g = g_ref[0]
    kl = (t[:, 0:F] + g[:, 0:F]) * inv6
    vl = (t[:, F:2 * F] + g[:, F:2 * F]) * inv6
    p = kl * vl
    ones_col = jnp.ones((n, 1), jnp.float32)
    kv = lax.dot_general(p, ones_col, (((0,), (0,)), ((), ())),
                         preferred_element_type=jnp.float32)
    hydra = qt_ref[0] * kv
    pt = lax.dot_general(wp_ref[...], hydra, (((1,), (0,)), ((), ())),
                         preferred_element_type=jnp.float32) + bp_ref[...]
    return lax.dot_general(wg1_ref[...], pt, (((1,), (0,)), ((), ())),
                           preferred_element_type=jnp.float32) + bg1_ref[...]


def _stage_c1_body(t1_ref, g1_ref, q1_ref, wp_ref, bp_ref, wg1_ref, bg1_ref,
                   z1o_ref, s1o_ref, ss1o_ref):
    n = q1_ref.shape[2]
    z1 = _z1_of(t1_ref, g1_ref, q1_ref, wp_ref, bp_ref, wg1_ref, bg1_ref, n)
    z1o_ref[0] = z1
    s1o_ref[0] = jnp.sum(z1, axis=1, keepdims=True)
    ss1o_ref[0] = jnp.sum(z1 * z1, axis=1, keepdims=True)


def _stage_c2_body(z1h_ref, s1h_ref, ss1h_ref, t2_ref, g2_ref, q2_ref,
                   wp_ref, bp_ref, wg1_ref, bg1_ref,
                   wg2_ref, bg2_ref, gm1_ref, beta1_ref, gm2_ref, beta2_ref,
                   out_ref, z1s_ref, z2s_ref, s1_ref, ss1_ref):
    nb = pl.num_programs(0)
    n = q2_ref.shape[2]
    ib = pl.program_id(0)
    z1 = _z1_of(t2_ref, g2_ref, q2_ref, wp_ref, bp_ref, wg1_ref, bg1_ref, n)
    z1s_ref[ib] = z1
    s1_ref[ib] = jnp.sum(z1, axis=1, keepdims=True)
    ss1_ref[ib] = jnp.sum(z1 * z1, axis=1, keepdims=True)

    @pl.when(ib == nb - 1)
    def _finalize():
        cnt = jnp.float32(2 * nb * n)
        m1 = (jnp.sum(s1_ref[...], axis=0)
              + jnp.sum(s1h_ref[...], axis=0)) / cnt
        v1 = (jnp.sum(ss1_ref[...], axis=0)
              + jnp.sum(ss1h_ref[...], axis=0)) / cnt - m1 * m1
        a1 = gm1_ref[...] / jnp.sqrt(v1 + 1e-5)
        c1 = beta1_ref[...] - m1 * a1
        s2 = jnp.zeros((OUTP, 1), jnp.float32)
        ss2 = jnp.zeros((OUTP, 1), jnp.float32)
        for i in range(2 * nb):
            z1i = z1h_ref[i] if i < nb else z1s_ref[i - nb]
            y1 = jnp.maximum(z1i * a1 + c1, 0.0)
            z2 = lax.dot_general(wg2_ref[...], y1, (((1,), (0,)), ((), ())),
                                 preferred_element_type=jnp.float32) + bg2_ref[...]
            z2s_ref[i] = z2
            s2 = s2 + jnp.sum(z2, axis=1, keepdims=True)
            ss2 = ss2 + jnp.sum(z2 * z2, axis=1, keepdims=True)
        m2 = s2 / cnt
        v2 = ss2 / cnt - m2 * m2
        a2 = gm2_ref[...] / jnp.sqrt(v2 + 1e-5)
        c2 = beta2_ref[...] - m2 * a2
        for i in range(z2s_ref.shape[0]):
            out_ref[i] = jnp.maximum(z2s_ref[i] * a2 + c2, 0.0)


def _sc_gather_sum(table, gidx, bn):
    info = plsc.get_sparse_core_info()
    nl = info.num_lanes
    nw = info.num_cores * info.num_subcores
    rw = bn // nw
    ch = 8
    nch = rw // ch
    d = 2 * F
    mesh = plsc.VectorSubcoreMesh(core_axis_name="c", subcore_axis_name="s")

    @functools.partial(
        pl.kernel,
        out_type=jax.ShapeDtypeStruct((bn, d), jnp.float32),
        mesh=mesh,
        scratch_types=[
            pltpu.VMEM((TOPK, rw), jnp.int32),
            [[pltpu.VMEM((ch, d), jnp.float32) for _ in range(TOPK)]
             for _ in range(2)],
            [pltpu.VMEM((ch, d), jnp.float32) for _ in range(2)],
            [pltpu.SemaphoreType.DMA for _ in range(2)],
            [pltpu.SemaphoreType.DMA for _ in range(2)],
        ],
    )
    def k(table_hbm, gidx_hbm, out_hbm, idx_v, bufs, obufs, gsems, osems):
        wid = lax.axis_index("s") * info.num_cores + lax.axis_index("c")
        base = wid * rw
        pltpu.sync_copy(gidx_hbm.at[pl.ds(0, TOPK), pl.ds(base, rw)], idx_v)

        def issue(c, s):
            for j in range(TOPK):
                pltpu.async_copy(
                    table_hbm.at[idx_v.at[j, pl.ds(c * ch, ch)]],
                    bufs[s][j], gsems[s])

        def drain_gathers(s):
            for j in range(TOPK):
                pltpu.make_async_copy(
                    table_hbm.at[idx_v.at[j, pl.ds(0, ch)]],
                    bufs[s][j], gsems[s]).wait()

        def wait_store(s):
            pltpu.make_async_copy(
                obufs[s], out_hbm.at[pl.ds(base, ch)], osems[s]).wait()

        def add_pass(s):
            def node(ni, carry):
                for si in range(d // nl):
                    sl = pl.ds(si * nl, nl)
                    acc = bufs[s][0][ni, sl]
                    for j in range(1, TOPK):
                        acc = acc + bufs[s][j][ni, sl]
                    obufs[s][ni, sl] = acc
                return carry
            lax.fori_loop(0, ch, node, 0)

        def body(i, carry):
            c0 = 2 * i
            drain_gathers(0)

            @pl.when(i > 0)
            def _ws0():
                wait_store(0)

            add_pass(0)

            @pl.when(c0 + 2 < nch)
            def _ig0():
                issue(c0 + 2, 0)

            pltpu.async_copy(obufs[0], out_hbm.at[pl.ds(base + c0 * ch, ch)],
                             osems[0])
            drain_gathers(1)

            @pl.when(i > 0)
            def _ws1():
                wait_store(1)

            add_pass(1)

            @pl.when(c0 + 3 < nch)
            def _ig1():
                issue(c0 + 3, 1)

            pltpu.async_copy(obufs[1], out_hbm.at[pl.ds(base + (c0 + 1) * ch,
                                                        ch)], osems[1])
            return carry

        issue(0, 0)
        issue(1, 1)
        lax.fori_loop(0, nch // 2, body, 0)
        wait_store(0)
        wait_store(1)

    return k(table, gidx)


def kernel(x, Wk, bk, Wq, bq, Wv, bv, Wp, bp, Wg1, bg1, Wg2, bg2, g1, beta1,
           g2, beta2):
    b, c, h, w = x.shape
    n = h * w
    bn = b * n
    xf = x.reshape(b, c, n)
    sel = (jnp.arange(F, dtype=jnp.int32)[:, None] // INTER
           == jnp.arange(HEADS, dtype=jnp.int32)[None, :]).astype(jnp.float32)

    full = lambda s: pl.BlockSpec(s, lambda i: (0,) * len(s))

    def stage_a(half):
        off = half * (b // 2)
        call = pl.pallas_call(
            _stage_a_body,
            grid=(b // 2,),
            in_specs=[
                pl.BlockSpec((1, c, n), lambda i: (i + off, 0, 0)),
                full((F, PLANE)), full((1, F)),
                full((F, PLANE)), full((F, 1)),
                full((F, PLANE)), full((1, F)),
                full((F, HEADS)),
            ],
            out_specs=[
                pl.BlockSpec((1, n, 2 * F), lambda i: (i, 0, 0)),
                pl.BlockSpec((1, F, n), lambda i: (i, 0, 0)),
                pl.BlockSpec((IDX_ROWS, n), lambda i: (0, i)),
            ],
            out_shape=[
                jax.ShapeDtypeStruct((b // 2, n, 2 * F), jnp.float32),
                jax.ShapeDtypeStruct((b // 2, F, n), jnp.float32),
                jax.ShapeDtypeStruct((IDX_ROWS, bn // 2), jnp.int32),
            ],
        )
        return call(xf, Wk, bk.reshape(1, F), Wq, bq.reshape(F, 1), Wv,
                    bv.reshape(1, F), sel)

    table1, qt1, gidx1 = stage_a(0)
    gsum1 = _sc_gather_sum(table1.reshape(bn // 2, 2 * F), gidx1, bn // 2)
    table2, qt2, gidx2 = stage_a(1)
    gsum2 = _sc_gather_sum(table2.reshape(bn // 2, 2 * F), gidx2, bn // 2)

    z1h, s1h, ss1h = pl.pallas_call(
        _stage_c1_body,
        grid=(b // 2,),
        in_specs=[
            pl.BlockSpec((1, n, 2 * F), lambda i: (i, 0, 0)),
            pl.BlockSpec((1, n, 2 * F), lambda i: (i, 0, 0)),
            pl.BlockSpec((1, F, n), lambda i: (i, 0, 0)),
            full((OUTP, F)), full((OUTP, 1)),
            full((OUTP, OUTP)), full((OUTP, 1)),
        ],
        out_specs=[
            pl.BlockSpec((1, OUTP, n), lambda i: (i, 0, 0)),
            pl.BlockSpec((1, OUTP, 1), lambda i: (i, 0, 0)),
            pl.BlockSpec((1, OUTP, 1), lambda i: (i, 0, 0)),
        ],
        out_shape=[
            jax.ShapeDtypeStruct((b // 2, OUTP, n), jnp.float32),
            jax.ShapeDtypeStruct((b // 2, OUTP, 1), jnp.float32),
            jax.ShapeDtypeStruct((b // 2, OUTP, 1), jnp.float32),
        ],
    )(table1, gsum1.reshape(b // 2, n, 2 * F), qt1, Wp, bp.reshape(OUTP, 1),
      Wg1, bg1.reshape(OUTP, 1))

    out = pl.pallas_call(
        _stage_c2_body,
        grid=(b // 2,),
        in_specs=[
            full((b // 2, OUTP, n)),
            full((b // 2, OUTP, 1)),
            full((b // 2, OUTP, 1)),
            pl.BlockSpec((1, n, 2 * F), lambda i: (i, 0, 0)),
            pl.BlockSpec((1, n, 2 * F), lambda i: (i, 0, 0)),
            pl.BlockSpec((1, F, n), lambda i: (i, 0, 0)),
            full((OUTP, F)), full((OUTP, 1)),
            full((OUTP, OUTP)), full((OUTP, 1)),
            full((OUTP, OUTP)), full((OUTP, 1)),
            full((OUTP, 1)), full((OUTP, 1)), full((OUTP, 1)), full((OUTP, 1)),
        ],
        out_specs=pl.BlockSpec((b, OUTP, n), lambda i: (0, 0, 0)),
        out_shape=jax.ShapeDtypeStruct((b, OUTP, n), jnp.float32),
        scratch_shapes=[
            pltpu.VMEM((b // 2, OUTP, n), jnp.float32),
            pltpu.VMEM((b, OUTP, n), jnp.float32),
            pltpu.VMEM((b // 2, OUTP, 1), jnp.float32),
            pltpu.VMEM((b // 2, OUTP, 1), jnp.float32),
        ],
    )(z1h, s1h, ss1h,
      table2, gsum2.reshape(b // 2, n, 2 * F), qt2,
      Wp, bp.reshape(OUTP, 1),
      Wg1, bg1.reshape(OUTP, 1), Wg2, bg2.reshape(OUTP, 1),
      g1.reshape(OUTP, 1), beta1.reshape(OUTP, 1), g2.reshape(OUTP, 1),
      beta2.reshape(OUTP, 1))
    return out.reshape(b, OUTP, h, w)
